# Initial kernel scaffold; baseline (speedup 1.0000x reference)
import jax
import jax.numpy as jnp
from jax.experimental import pallas as pl


def kernel(patch_embs, edge_index, edge_attr, W1, root1, bias1, W2, root2, bias2, W3, root3, bias3, g1, beta1, g2, beta2, g3, beta3):
    raise NotImplementedError("write your pallas kernel here")


# speedup vs baseline: 6.2317x; 6.2317x over previous
import dataclasses
import functools

import jax
import jax.numpy as jnp
from jax import lax
from jax.experimental import pallas as pl
from jax.experimental.pallas import tpu as pltpu
from jax.experimental.pallas import tpu_sc as plsc

KS = 5
DIM = 3
K = KS ** DIM
F = 32
N = 10000
E = 160000
S = 8

NC = 2
NSUB = 16
NW = NC * NSUB
CHUNK = 128
CPW = 40
EPW = CHUNK * CPW
E_PAD = EPW * NW
ROW = 48
OCHUNK = 80
NOCHUNK = N // OCHUNK

_PREP_R = 128
_PREP_C = E_PAD // _PREP_R


def _prep_body(attr_ref, src_ref, valid_ref, gidx_ref, basis_ref):
    fr, lo = [], []
    for d in range(DIM):
        v = attr_ref[d] * float(KS - 1)
        lf = jnp.floor(v)
        fr.append(v - lf)
        lo.append(lf.astype(jnp.int32))
    src = src_ref[...]
    valid = valid_ref[...]
    for s in range(S):
        b = valid
        kk = src * K
        stride = 1
        for d in range(DIM):
            bit = (s >> d) & 1
            b = b * (fr[d] if bit else (1.0 - fr[d]))
            kk = kk + (lo[d] + bit) * stride
            stride *= KS
        gidx_ref[s] = kk
        basis_ref[s] = b


def _prep(attr3, src2, valid2):
    return pl.pallas_call(
        _prep_body,
        out_shape=(
            jax.ShapeDtypeStruct((S, _PREP_R, _PREP_C), jnp.int32),
            jax.ShapeDtypeStruct((S, _PREP_R, _PREP_C), jnp.float32),
        ),
    )(attr3, src2, valid2)


_YBLK = 400


def _ymm_body(x_ref, w_ref, y_ref):
    y_ref[...] = jnp.dot(x_ref[...], w_ref[...],
                         preferred_element_type=jnp.float32)


def _ymm(x, w2d):
    return pl.pallas_call(
        _ymm_body,
        grid=(N // _YBLK,),
        in_specs=[
            pl.BlockSpec((_YBLK, F), lambda i: (i, 0)),
            pl.BlockSpec((F, K * F), lambda i: (0, 0)),
        ],
        out_specs=pl.BlockSpec((_YBLK, K * F), lambda i: (i, 0)),
        out_shape=jax.ShapeDtypeStruct((N, K * F), jnp.float32),
    )(x, w2d)


def _sc_body(yt, gidxT, basisT, dst1, out, idx_v, bas_v, dst_v, rows_v,
             msg_v, zero_v, acc_sh, sem_m, sem_g):
    cid = lax.axis_index("c")
    sid = lax.axis_index("s")
    wid = cid * NSUB + sid

    zeros16 = jnp.zeros((16,), jnp.float32)

    @pl.loop(0, OCHUNK)
    def _(r):
        for j in range(ROW // 16):
            zero_v[r, pl.ds(j * 16, 16)] = zeros16

    @pl.loop(0, CHUNK)
    def _(r):
        for j in range(ROW // 16):
            msg_v[r, pl.ds(j * 16, 16)] = zeros16

    @pl.loop(0, pl.cdiv(NOCHUNK, NSUB))
    def _(t):
        c = t * NSUB + sid

        @pl.when(c < NOCHUNK)
        def _():
            pltpu.sync_copy(zero_v, acc_sh.at[pl.ds(c * OCHUNK, OCHUNK)])

    plsc.subcore_barrier()

    lane = lax.iota(jnp.int32, 16)

    @pl.loop(0, CPW)
    def _(i):
        base = wid * EPW + i * CHUNK
        cp1 = pltpu.async_copy(gidxT.at[:, pl.ds(base, CHUNK)], idx_v, sem_m)
        cp2 = pltpu.async_copy(basisT.at[:, pl.ds(base, CHUNK)], bas_v, sem_m)
        cp3 = pltpu.async_copy(dst1.at[pl.ds(base, CHUNK)], dst_v, sem_m)
        cp1.wait()
        cp2.wait()
        cp3.wait()
        gcps = [
            pltpu.async_copy(yt.at[idx_v.at[s]],
                             rows_v.at[pl.ds(s * CHUNK, CHUNK)], sem_g)
            for s in range(S)
        ]
        for cp in gcps:
            cp.wait()

        @pl.loop(0, CHUNK // 16)
        def _(g):
            e16 = g * 16
            row_e = e16 + lane
            bs = [bas_v[s, pl.ds(e16, 16)] for s in range(S)]
            deg = bs[0]
            for s in range(1, S):
                deg = deg + bs[s]
            rrows = [s * CHUNK + row_e for s in range(S)]
            for f in range(F):
                fz = jnp.full((16,), f, jnp.int32)
                acc = bs[0] * plsc.load_gather(rows_v, [rrows[0], fz])
                for s in range(1, S):
                    acc = acc + bs[s] * plsc.load_gather(rows_v, [rrows[s], fz])
                plsc.store_scatter(msg_v, [row_e, fz], acc)
            plsc.store_scatter(msg_v, [row_e, jnp.full((16,), F, jnp.int32)],
                               deg)

        pltpu.sync_copy(msg_v, acc_sh.at[dst_v], add=True)

    plsc.subcore_barrier()

    @pl.loop(0, pl.cdiv(NOCHUNK, NSUB))
    def _(t):
        c = t * NSUB + sid

        @pl.when(c < NOCHUNK)
        def _():
            pltpu.sync_copy(acc_sh.at[pl.ds(c * OCHUNK, OCHUNK)],
                            out.at[cid, pl.ds(c * OCHUNK, OCHUNK)])


def _spline_sc(yt, gidxT, basisT, dst1):
    mesh = plsc.VectorSubcoreMesh(core_axis_name="c", subcore_axis_name="s")
    cp = pltpu.CompilerParams()
    if "needs_layout_passes" in pltpu.CompilerParams.__dataclass_fields__:
        cp = dataclasses.replace(cp, needs_layout_passes=False)
    if "use_tc_tiling_on_sc" in pltpu.CompilerParams.__dataclass_fields__:
        cp = dataclasses.replace(cp, use_tc_tiling_on_sc=False)
    fn = pl.kernel(
        _sc_body,
        out_type=jax.ShapeDtypeStruct((NC, N, ROW), jnp.float32),
        mesh=mesh,
        scratch_types=[
            pltpu.VMEM((S, CHUNK), jnp.int32),
            pltpu.VMEM((S, CHUNK), jnp.float32),
            pltpu.VMEM((CHUNK,), jnp.int32),
            pltpu.VMEM((S * CHUNK, F), jnp.float32),
            pltpu.VMEM((CHUNK, ROW), jnp.float32),
            pltpu.VMEM((OCHUNK, ROW), jnp.float32),
            pltpu.VMEM_SHARED((N, ROW), jnp.float32),
            pltpu.SemaphoreType.DMA,
            pltpu.SemaphoreType.DMA,
        ],
        compiler_params=cp,
    )
    return fn(yt, gidxT, basisT, dst1)


def _post_body(parts_ref, x_ref, root_ref, bias_ref, g_ref, beta_ref, o_ref,
               *, leaky):
    p = parts_ref[0] + parts_ref[1]
    conv = p[:, :F]
    deg = p[:, F:F + 1]
    conv = conv / jnp.maximum(deg, 1.0)
    x = x_ref[...]
    conv = conv + jnp.dot(x, root_ref[...],
                          preferred_element_type=jnp.float32) + bias_ref[...]
    if leaky:
        conv = jnp.where(conv >= 0.0, conv, 0.01 * conv)
    h = conv + x
    m = jnp.mean(h, axis=0, keepdims=True)
    v = jnp.mean((h - m) ** 2, axis=0, keepdims=True)
    o_ref[...] = g_ref[...] * (h - m) / jnp.sqrt(v + 1e-5) + beta_ref[...]


def _post(parts, x, root, bias, g, beta, leaky):
    return pl.pallas_call(
        functools.partial(_post_body, leaky=leaky),
        out_shape=jax.ShapeDtypeStruct((N, F), jnp.float32),
    )(parts, x, root.reshape(F, F), bias.reshape(1, F), g.reshape(1, F),
      beta.reshape(1, F))


def kernel(patch_embs, edge_index, edge_attr, W1, root1, bias1, W2, root2,
           bias2, W3, root3, bias3, g1, beta1, g2, beta2, g3, beta3):
    src = edge_index[0]
    dst = edge_index[1]
    pad = E_PAD - E
    srcp = jnp.concatenate([src, jnp.zeros((pad,), jnp.int32)])
    dstp = jnp.concatenate([dst, jnp.zeros((pad,), jnp.int32)])
    attrp = jnp.concatenate(
        [edge_attr.T, jnp.zeros((DIM, pad), jnp.float32)], axis=1)
    validp = jnp.concatenate(
        [jnp.ones((E,), jnp.float32), jnp.zeros((pad,), jnp.float32)])

    gidxT, basisT = _prep(attrp.reshape(DIM, _PREP_R, _PREP_C),
                          srcp.reshape(_PREP_R, _PREP_C),
                          validp.reshape(_PREP_R, _PREP_C))
    gidxT = gidxT.reshape(S, E_PAD)
    basisT = basisT.reshape(S, E_PAD)

    x = patch_embs
    layers = ((W1, root1, bias1, g1, beta1, True),
              (W2, root2, bias2, g2, beta2, True),
              (W3, root3, bias3, g3, beta3, False))
    for (W, root, bias, g, beta, leaky) in layers:
        w2d = jnp.transpose(W, (1, 0, 2)).reshape(F, K * F)
        y = _ymm(x, w2d).reshape(N * K, F)
        parts = _spline_sc(y, gidxT, basisT, dstp)
        x = _post(parts, x, root, bias, g, beta, leaky)
    return x


# speedup vs baseline: 7.2134x; 1.1575x over previous
import dataclasses
import functools

import jax
import jax.numpy as jnp
from jax import lax
from jax.experimental import pallas as pl
from jax.experimental.pallas import tpu as pltpu
from jax.experimental.pallas import tpu_sc as plsc

KS = 5
DIM = 3
K = KS ** DIM
F = 32
N = 10000
E = 160000
S = 8

NC = 2
NSUB = 16
NW = NC * NSUB
CHUNK = 128
CPW = 40
EPW = CHUNK * CPW
E_PAD = EPW * NW
ROW = 48
OCHUNK = 80
NOCHUNK = N // OCHUNK

_PREP_R = 128
_PREP_C = E_PAD // _PREP_R


def _prep_body(attr_ref, src_ref, valid_ref, gidx_ref, basis_ref):
    fr, lo = [], []
    for d in range(DIM):
        v = attr_ref[d] * float(KS - 1)
        lf = jnp.floor(v)
        fr.append(v - lf)
        lo.append(lf.astype(jnp.int32))
    src = src_ref[...]
    valid = valid_ref[...]
    for s in range(S):
        b = valid
        kk = src * K
        stride = 1
        for d in range(DIM):
            bit = (s >> d) & 1
            b = b * (fr[d] if bit else (1.0 - fr[d]))
            kk = kk + (lo[d] + bit) * stride
            stride *= KS
        gidx_ref[s] = kk
        basis_ref[s] = b


def _prep(attr3, src2, valid2):
    return pl.pallas_call(
        _prep_body,
        out_shape=(
            jax.ShapeDtypeStruct((S, _PREP_R, _PREP_C), jnp.int32),
            jax.ShapeDtypeStruct((S, _PREP_R, _PREP_C), jnp.float32),
        ),
    )(attr3, src2, valid2)


_YBLK = 400


def _ymm_body(x_ref, w_ref, y_ref):
    y_ref[...] = jnp.dot(x_ref[...], w_ref[...],
                         preferred_element_type=jnp.float32)


def _ymm(x, w2d):
    return pl.pallas_call(
        _ymm_body,
        grid=(N // _YBLK,),
        in_specs=[
            pl.BlockSpec((_YBLK, F), lambda i: (i, 0)),
            pl.BlockSpec((F, K * F), lambda i: (0, 0)),
        ],
        out_specs=pl.BlockSpec((_YBLK, K * F), lambda i: (i, 0)),
        out_shape=jax.ShapeDtypeStruct((N, K * F), jnp.float32),
    )(x, w2d)


def _sc_body(yt, gidxT, basisT, dst2, out, idx_v, bas_v, dst_v, rows_v,
             msg_v, zero_v, acc_sh, sem_m, sem_g, sem_s):
    cid = lax.axis_index("c")
    sid = lax.axis_index("s")
    wid = cid * NSUB + sid

    zeros16 = jnp.zeros((16,), jnp.float32)

    @pl.loop(0, OCHUNK)
    def _(r):
        for j in range(ROW // 16):
            zero_v[r, pl.ds(j * 16, 16)] = zeros16

    @pl.loop(0, CHUNK)
    def _(r):
        for j in range(ROW // 16):
            for p in range(2):
                msg_v[p, r, pl.ds(j * 16, 16)] = zeros16

    @pl.loop(0, pl.cdiv(NOCHUNK, NSUB))
    def _(t):
        c = t * NSUB + sid

        @pl.when(c < NOCHUNK)
        def _():
            pltpu.sync_copy(zero_v, acc_sh.at[pl.ds(c * OCHUNK, OCHUNK)])

    pltpu.async_copy(dst2.at[pl.ds(wid * CPW, CPW)], dst_v, sem_m).wait()
    plsc.subcore_barrier()

    lane = lax.iota(jnp.int32, 16)

    def meta_start(i, p):
        base = wid * EPW + i * CHUNK
        pltpu.async_copy(gidxT.at[:, pl.ds(base, CHUNK)], idx_v.at[p], sem_m)
        pltpu.async_copy(basisT.at[:, pl.ds(base, CHUNK)], bas_v.at[p], sem_m)

    def meta_wait(i, p):
        base = wid * EPW + i * CHUNK
        pltpu.make_async_copy(
            gidxT.at[:, pl.ds(base, CHUNK)], idx_v.at[p], sem_m).wait()
        pltpu.make_async_copy(
            basisT.at[:, pl.ds(base, CHUNK)], bas_v.at[p], sem_m).wait()

    def gather_start(p):
        for s in range(S):
            pltpu.async_copy(yt.at[idx_v.at[p, s]],
                             rows_v.at[p, pl.ds(s * CHUNK, CHUNK)], sem_g)

    def gather_wait(p):
        for s in range(S):
            pltpu.make_async_copy(yt.at[idx_v.at[p, s]],
                                  rows_v.at[p, pl.ds(s * CHUNK, CHUNK)],
                                  sem_g).wait()

    def compute(i, p):
        rows_p = rows_v.at[p]
        msg_p = msg_v.at[p]

        @pl.loop(0, CHUNK // 16)
        def _(g):
            e16 = g * 16
            row_e = e16 + lane
            bs = [bas_v[p, s, pl.ds(e16, 16)] for s in range(S)]
            deg = bs[0]
            for s in range(1, S):
                deg = deg + bs[s]
            rrows = [s * CHUNK + row_e for s in range(S)]
            for f in range(F):
                fz = jnp.full((16,), f, jnp.int32)
                acc = bs[0] * plsc.load_gather(rows_p, [rrows[0], fz])
                for s in range(1, S):
                    acc = acc + bs[s] * plsc.load_gather(rows_p, [rrows[s], fz])
                plsc.store_scatter(msg_p, [row_e, fz], acc)
            plsc.store_scatter(msg_p, [row_e, jnp.full((16,), F, jnp.int32)],
                               deg)

    def scatter_start(i, p):
        pltpu.async_copy(msg_v.at[p], acc_sh.at[dst_v.at[i]], sem_s, add=True)

    def scatter_wait(i, p):
        pltpu.make_async_copy(msg_v.at[p], acc_sh.at[dst_v.at[i]],
                              sem_s).wait()

    meta_start(0, 0)

    def half(i, p):
        im1 = i - 1

        @pl.when((i >= 1) & (im1 < CPW))
        def _():
            gather_wait(1 - p)

        @pl.when(i < CPW)
        def _():
            meta_wait(i, p)
            gather_start(p)

        @pl.when((i >= 1) & (im1 < CPW))
        def _():
            @pl.when(im1 >= 2)
            def _():
                scatter_wait(im1 - 2, 1 - p)

            compute(im1, 1 - p)
            scatter_start(im1, 1 - p)

        @pl.when(i + 1 < CPW)
        def _():
            meta_start(i + 1, 1 - p)

    @pl.loop(0, CPW + 2, step=2)
    def _(i0):
        half(i0, 0)
        half(i0 + 1, 1)

    scatter_wait(CPW - 2, 0)
    scatter_wait(CPW - 1, 1)

    plsc.subcore_barrier()

    @pl.loop(0, pl.cdiv(NOCHUNK, NSUB))
    def _(t):
        c = t * NSUB + sid

        @pl.when(c < NOCHUNK)
        def _():
            pltpu.sync_copy(acc_sh.at[pl.ds(c * OCHUNK, OCHUNK)],
                            out.at[cid, pl.ds(c * OCHUNK, OCHUNK)])


def _spline_sc(yt, gidxT, basisT, dst2):
    mesh = plsc.VectorSubcoreMesh(core_axis_name="c", subcore_axis_name="s")
    cp = pltpu.CompilerParams()
    if "needs_layout_passes" in pltpu.CompilerParams.__dataclass_fields__:
        cp = dataclasses.replace(cp, needs_layout_passes=False)
    if "use_tc_tiling_on_sc" in pltpu.CompilerParams.__dataclass_fields__:
        cp = dataclasses.replace(cp, use_tc_tiling_on_sc=False)
    fn = pl.kernel(
        _sc_body,
        out_type=jax.ShapeDtypeStruct((NC, N, ROW), jnp.float32),
        mesh=mesh,
        scratch_types=[
            pltpu.VMEM((2, S, CHUNK), jnp.int32),
            pltpu.VMEM((2, S, CHUNK), jnp.float32),
            pltpu.VMEM((CPW, CHUNK), jnp.int32),
            pltpu.VMEM((2, S * CHUNK, F), jnp.float32),
            pltpu.VMEM((2, CHUNK, ROW), jnp.float32),
            pltpu.VMEM((OCHUNK, ROW), jnp.float32),
            pltpu.VMEM_SHARED((N, ROW), jnp.float32),
            pltpu.SemaphoreType.DMA,
            pltpu.SemaphoreType.DMA,
            pltpu.SemaphoreType.DMA,
        ],
        compiler_params=cp,
    )
    return fn(yt, gidxT, basisT, dst2)


def _post_body(parts_ref, x_ref, root_ref, bias_ref, g_ref, beta_ref, o_ref,
               *, leaky):
    p = parts_ref[0] + parts_ref[1]
    conv = p[:, :F]
    deg = p[:, F:F + 1]
    conv = conv / jnp.maximum(deg, 1.0)
    x = x_ref[...]
    conv = conv + jnp.dot(x, root_ref[...],
                          preferred_element_type=jnp.float32) + bias_ref[...]
    if leaky:
        conv = jnp.where(conv >= 0.0, conv, 0.01 * conv)
    h = conv + x
    m = jnp.mean(h, axis=0, keepdims=True)
    v = jnp.mean((h - m) ** 2, axis=0, keepdims=True)
    o_ref[...] = g_ref[...] * (h - m) / jnp.sqrt(v + 1e-5) + beta_ref[...]


def _post(parts, x, root, bias, g, beta, leaky):
    return pl.pallas_call(
        functools.partial(_post_body, leaky=leaky),
        out_shape=jax.ShapeDtypeStruct((N, F), jnp.float32),
    )(parts, x, root.reshape(F, F), bias.reshape(1, F), g.reshape(1, F),
      beta.reshape(1, F))


def kernel(patch_embs, edge_index, edge_attr, W1, root1, bias1, W2, root2,
           bias2, W3, root3, bias3, g1, beta1, g2, beta2, g3, beta3):
    src = edge_index[0]
    dst = edge_index[1]
    pad = E_PAD - E
    srcp = jnp.concatenate([src, jnp.zeros((pad,), jnp.int32)])
    dstp = jnp.concatenate([dst, jnp.zeros((pad,), jnp.int32)])
    attrp = jnp.concatenate(
        [edge_attr.T, jnp.zeros((DIM, pad), jnp.float32)], axis=1)
    validp = jnp.concatenate(
        [jnp.ones((E,), jnp.float32), jnp.zeros((pad,), jnp.float32)])

    gidxT, basisT = _prep(attrp.reshape(DIM, _PREP_R, _PREP_C),
                          srcp.reshape(_PREP_R, _PREP_C),
                          validp.reshape(_PREP_R, _PREP_C))
    gidxT = gidxT.reshape(S, E_PAD)
    basisT = basisT.reshape(S, E_PAD)
    dst2 = dstp.reshape(E_PAD // CHUNK, CHUNK)

    x = patch_embs
    layers = ((W1, root1, bias1, g1, beta1, True),
              (W2, root2, bias2, g2, beta2, True),
              (W3, root3, bias3, g3, beta3, False))
    for (W, root, bias, g, beta, leaky) in layers:
        w2d = jnp.transpose(W, (1, 0, 2)).reshape(F, K * F)
        y = _ymm(x, w2d).reshape(N * K, F)
        parts = _spline_sc(y, gidxT, basisT, dst2)
        x = _post(parts, x, root, bias, g, beta, leaky)
    return x


# speedup vs baseline: 7.9879x; 1.1074x over previous
import dataclasses
import functools

import jax
import jax.numpy as jnp
from jax import lax
from jax.experimental import pallas as pl
from jax.experimental.pallas import tpu as pltpu
from jax.experimental.pallas import tpu_sc as plsc

KS = 5
DIM = 3
K = KS ** DIM
F = 32
N = 10000
E = 160000
S = 8

NC = 2
NSUB = 16
NW = NC * NSUB
CHUNK = 128
CPW = 40
EPW = CHUNK * CPW
E_PAD = EPW * NW
ROW = 48
OCHUNK = 80
NOCHUNK = N // OCHUNK

_PREP_R = 128
_PREP_C = E_PAD // _PREP_R


def _prep_body(attr_ref, src_ref, valid_ref, gidx_ref, basis_ref):
    fr, lo = [], []
    for d in range(DIM):
        v = attr_ref[d] * float(KS - 1)
        lf = jnp.floor(v)
        fr.append(v - lf)
        lo.append(lf.astype(jnp.int32))
    src = src_ref[...]
    valid = valid_ref[...]
    for s in range(S):
        b = valid
        kk = src * K
        stride = 1
        for d in range(DIM):
            bit = (s >> d) & 1
            b = b * (fr[d] if bit else (1.0 - fr[d]))
            kk = kk + (lo[d] + bit) * stride
            stride *= KS
        gidx_ref[s] = kk
        basis_ref[s] = b


def _prep(attr3, src2, valid2):
    return pl.pallas_call(
        _prep_body,
        out_shape=(
            jax.ShapeDtypeStruct((S, _PREP_R, _PREP_C), jnp.int32),
            jax.ShapeDtypeStruct((S, _PREP_R, _PREP_C), jnp.float32),
        ),
    )(attr3, src2, valid2)


_YBLK = 400


def _ymm_body(x_ref, w_ref, y_ref):
    y_ref[...] = jnp.dot(x_ref[...], w_ref[...],
                         preferred_element_type=jnp.float32)


def _ymm(x, w2d):
    return pl.pallas_call(
        _ymm_body,
        grid=(N // _YBLK,),
        in_specs=[
            pl.BlockSpec((_YBLK, F), lambda i: (i, 0)),
            pl.BlockSpec((F, K * F), lambda i: (0, 0)),
        ],
        out_specs=pl.BlockSpec((_YBLK, K * F), lambda i: (i, 0)),
        out_shape=jax.ShapeDtypeStruct((N, K * F), jnp.float32),
    )(x, w2d)


def _sc_body(yt, gidxT, basisT, dst2, out, idx_v, bas_v, dst_v, rows_v,
             msg_v, zero_v, acc_sh, sem_m, sem_g, sem_s):
    cid = lax.axis_index("c")
    sid = lax.axis_index("s")
    wid = cid * NSUB + sid

    zeros16 = jnp.zeros((16,), jnp.float32)

    @pl.loop(0, OCHUNK)
    def _(r):
        for j in range(ROW // 16):
            zero_v[r, pl.ds(j * 16, 16)] = zeros16

    @pl.loop(0, CHUNK)
    def _(r):
        for j in range(ROW // 16):
            for p in range(2):
                msg_v[p, r, pl.ds(j * 16, 16)] = zeros16

    @pl.loop(0, pl.cdiv(NOCHUNK, NSUB))
    def _(t):
        c = t * NSUB + sid

        @pl.when(c < NOCHUNK)
        def _():
            pltpu.sync_copy(zero_v, acc_sh.at[pl.ds(c * OCHUNK, OCHUNK)])

    pltpu.async_copy(dst2.at[pl.ds(wid * CPW, CPW)], dst_v, sem_m).wait()
    plsc.subcore_barrier()

    lane = lax.iota(jnp.int32, 16)

    def meta_start(i, p):
        base = wid * EPW + i * CHUNK
        pltpu.async_copy(gidxT.at[:, pl.ds(base, CHUNK)], idx_v.at[p], sem_m)
        pltpu.async_copy(basisT.at[:, pl.ds(base, CHUNK)], bas_v.at[p], sem_m)

    def meta_wait(i, p):
        base = wid * EPW + i * CHUNK
        pltpu.make_async_copy(
            gidxT.at[:, pl.ds(base, CHUNK)], idx_v.at[p], sem_m).wait()
        pltpu.make_async_copy(
            basisT.at[:, pl.ds(base, CHUNK)], bas_v.at[p], sem_m).wait()

    def gather_start(p):
        for s in range(S):
            pltpu.async_copy(yt.at[idx_v.at[p, s]],
                             rows_v.at[p, pl.ds(s * CHUNK, CHUNK)], sem_g)

    def gather_wait(p):
        for s in range(S):
            pltpu.make_async_copy(yt.at[idx_v.at[p, s]],
                                  rows_v.at[p, pl.ds(s * CHUNK, CHUNK)],
                                  sem_g).wait()

    def tree_sum(vals):
        vals = list(vals)
        while len(vals) > 1:
            vals = [a + b for a, b in zip(vals[::2], vals[1::2])]
        return vals[0]

    def compute(i, p):
        rows_p = rows_v.at[p]
        msg_p = msg_v.at[p]

        @plsc.parallel_loop(0, CHUNK // 16)
        def _(g):
            e16 = g * 16
            row_e = e16 + lane
            bs = [bas_v[p, s, pl.ds(e16, 16)] for s in range(S)]
            deg = tree_sum(bs)
            rrows = [s * CHUNK + row_e for s in range(S)]
            for f0 in range(0, F, 8):
                accs = []
                for f in range(f0, f0 + 8):
                    fz = jnp.full((16,), f, jnp.int32)
                    accs.append(tree_sum(
                        bs[s] * plsc.load_gather(rows_p, [rrows[s], fz])
                        for s in range(S)))
                for j, f in enumerate(range(f0, f0 + 8)):
                    plsc.store_scatter(
                        msg_p, [row_e, jnp.full((16,), f, jnp.int32)], accs[j])
            plsc.store_scatter(msg_p, [row_e, jnp.full((16,), F, jnp.int32)],
                               deg)

    def scatter_start(i, p):
        pltpu.async_copy(msg_v.at[p], acc_sh.at[dst_v.at[i]], sem_s, add=True)

    def scatter_wait(i, p):
        pltpu.make_async_copy(msg_v.at[p], acc_sh.at[dst_v.at[i]],
                              sem_s).wait()

    meta_start(0, 0)

    def half(i, p):
        im1 = i - 1

        @pl.when((i >= 1) & (im1 < CPW))
        def _():
            gather_wait(1 - p)

        @pl.when(i < CPW)
        def _():
            meta_wait(i, p)
            gather_start(p)

        @pl.when((i >= 1) & (im1 < CPW))
        def _():
            @pl.when(im1 >= 2)
            def _():
                scatter_wait(im1 - 2, 1 - p)

            compute(im1, 1 - p)
            scatter_start(im1, 1 - p)

        @pl.when(i + 1 < CPW)
        def _():
            meta_start(i + 1, 1 - p)

    @pl.loop(0, CPW + 2, step=2)
    def _(i0):
        half(i0, 0)
        half(i0 + 1, 1)

    scatter_wait(CPW - 2, 0)
    scatter_wait(CPW - 1, 1)

    plsc.subcore_barrier()

    @pl.loop(0, pl.cdiv(NOCHUNK, NSUB))
    def _(t):
        c = t * NSUB + sid

        @pl.when(c < NOCHUNK)
        def _():
            pltpu.sync_copy(acc_sh.at[pl.ds(c * OCHUNK, OCHUNK)],
                            out.at[cid, pl.ds(c * OCHUNK, OCHUNK)])


def _spline_sc(yt, gidxT, basisT, dst2):
    mesh = plsc.VectorSubcoreMesh(core_axis_name="c", subcore_axis_name="s")
    cp = pltpu.CompilerParams()
    if "needs_layout_passes" in pltpu.CompilerParams.__dataclass_fields__:
        cp = dataclasses.replace(cp, needs_layout_passes=False)
    if "use_tc_tiling_on_sc" in pltpu.CompilerParams.__dataclass_fields__:
        cp = dataclasses.replace(cp, use_tc_tiling_on_sc=False)
    fn = pl.kernel(
        _sc_body,
        out_type=jax.ShapeDtypeStruct((NC, N, ROW), jnp.float32),
        mesh=mesh,
        scratch_types=[
            pltpu.VMEM((2, S, CHUNK), jnp.int32),
            pltpu.VMEM((2, S, CHUNK), jnp.float32),
            pltpu.VMEM((CPW, CHUNK), jnp.int32),
            pltpu.VMEM((2, S * CHUNK, F), jnp.float32),
            pltpu.VMEM((2, CHUNK, ROW), jnp.float32),
            pltpu.VMEM((OCHUNK, ROW), jnp.float32),
            pltpu.VMEM_SHARED((N, ROW), jnp.float32),
            pltpu.SemaphoreType.DMA,
            pltpu.SemaphoreType.DMA,
            pltpu.SemaphoreType.DMA,
        ],
        compiler_params=cp,
    )
    return fn(yt, gidxT, basisT, dst2)


def _post_body(parts_ref, x_ref, root_ref, bias_ref, g_ref, beta_ref, o_ref,
               *, leaky):
    p = parts_ref[0] + parts_ref[1]
    conv = p[:, :F]
    deg = p[:, F:F + 1]
    conv = conv / jnp.maximum(deg, 1.0)
    x = x_ref[...]
    conv = conv + jnp.dot(x, root_ref[...],
                          preferred_element_type=jnp.float32) + bias_ref[...]
    if leaky:
        conv = jnp.where(conv >= 0.0, conv, 0.01 * conv)
    h = conv + x
    m = jnp.mean(h, axis=0, keepdims=True)
    v = jnp.mean((h - m) ** 2, axis=0, keepdims=True)
    o_ref[...] = g_ref[...] * (h - m) / jnp.sqrt(v + 1e-5) + beta_ref[...]


def _post(parts, x, root, bias, g, beta, leaky):
    return pl.pallas_call(
        functools.partial(_post_body, leaky=leaky),
        out_shape=jax.ShapeDtypeStruct((N, F), jnp.float32),
    )(parts, x, root.reshape(F, F), bias.reshape(1, F), g.reshape(1, F),
      beta.reshape(1, F))


def kernel(patch_embs, edge_index, edge_attr, W1, root1, bias1, W2, root2,
           bias2, W3, root3, bias3, g1, beta1, g2, beta2, g3, beta3):
    src = edge_index[0]
    dst = edge_index[1]
    pad = E_PAD - E
    srcp = jnp.concatenate([src, jnp.zeros((pad,), jnp.int32)])
    dstp = jnp.concatenate([dst, jnp.zeros((pad,), jnp.int32)])
    attrp = jnp.concatenate(
        [edge_attr.T, jnp.zeros((DIM, pad), jnp.float32)], axis=1)
    validp = jnp.concatenate(
        [jnp.ones((E,), jnp.float32), jnp.zeros((pad,), jnp.float32)])

    gidxT, basisT = _prep(attrp.reshape(DIM, _PREP_R, _PREP_C),
                          srcp.reshape(_PREP_R, _PREP_C),
                          validp.reshape(_PREP_R, _PREP_C))
    gidxT = gidxT.reshape(S, E_PAD)
    basisT = basisT.reshape(S, E_PAD)
    dst2 = dstp.reshape(E_PAD // CHUNK, CHUNK)

    x = patch_embs
    layers = ((W1, root1, bias1, g1, beta1, True),
              (W2, root2, bias2, g2, beta2, True),
              (W3, root3, bias3, g3, beta3, False))
    for (W, root, bias, g, beta, leaky) in layers:
        w2d = jnp.transpose(W, (1, 0, 2)).reshape(F, K * F)
        y = _ymm(x, w2d).reshape(N * K, F)
        parts = _spline_sc(y, gidxT, basisT, dst2)
        x = _post(parts, x, root, bias, g, beta, leaky)
    return x
